# baseline (speedup 1.0000x reference)
import jax
import jax.numpy as jnp
from jax import lax
from jax.experimental import pallas as pl
from jax.experimental.pallas import tpu as pltpu
from jax.experimental.pallas import tpu_sc as plsc

_SEQ = 128
_CLS = 101
_SEP = 102
_TOK = 4096
_B = 16
_D = 128
_HALF = 64


def _body(ids_hbm, cu_hbm, emb_hbm,
          word_hbm, mask_hbm, type_hbm, emb_out_hbm,
          cu_v, idx_v, rows_v, gids_v, word_v, mask_v, type_v,
          sem_a, sem_b, sem_ids, sem_out):
    b = lax.axis_index("s")

    pltpu.sync_copy(cu_hbm, cu_v.at[pl.ds(0, _B + 1)])
    lane = lax.iota(jnp.int32, 16)
    cuv = cu_v[pl.ds(b, 16)]
    start = cuv[0]
    seglen = jnp.minimum(cuv[1] - start, _SEQ - 2)

    for kk in range(_SEQ // 16):
        jj = lane + kk * 16
        idxc = jnp.minimum(jnp.maximum(start + jj - 1, 0), _TOK - 1)
        idx_v[pl.ds(kk * 16, 16)] = idxc

    cp_a = pltpu.async_copy(emb_hbm.at[idx_v.at[pl.ds(0, _HALF)]],
                            rows_v.at[pl.ds(0, _HALF), :], sem_a)
    cp_b = pltpu.async_copy(emb_hbm.at[idx_v.at[pl.ds(_HALF, _HALF)]],
                            rows_v.at[pl.ds(_HALF, _HALF), :], sem_b)
    cp_ids = pltpu.async_copy(ids_hbm.at[idx_v], gids_v, sem_ids)

    for kk in range(_SEQ // 16):
        jj = lane + kk * 16
        mask_v[pl.ds(kk * 16, 16)] = jnp.where(jj <= seglen + 1, 1, 0)
        type_v[pl.ds(kk * 16, 16)] = jj - jj
    cp_mask = pltpu.async_copy(mask_v, mask_hbm.at[b], sem_out)
    cp_type = pltpu.async_copy(type_v, type_hbm.at[b], sem_out)

    cp_ids.wait()
    for kk in range(_SEQ // 16):
        jj = lane + kk * 16
        g = gids_v[pl.ds(kk * 16, 16)]
        tok = (jj >= 1) & (jj <= seglen)
        w = jnp.where(jj == 0, _CLS,
                      jnp.where(tok, g,
                                jnp.where(jj == seglen + 1, _SEP, 0)))
        word_v[pl.ds(kk * 16, 16)] = w
    cp_word = pltpu.async_copy(word_v, word_hbm.at[b], sem_out)

    zf = jnp.zeros((16,), jnp.float32)

    def _zero_row(r, carry):
        for cc in range(_D // 16):
            rows_v[r, pl.ds(cc * 16, 16)] = zf
        return carry

    hi = seglen + 1
    cp_a.wait()
    for cc in range(_D // 16):
        rows_v[0, pl.ds(cc * 16, 16)] = zf
    lax.fori_loop(jnp.minimum(hi, _HALF), _HALF, _zero_row, 0)
    cp_o1 = pltpu.async_copy(rows_v.at[pl.ds(0, _HALF), :],
                             emb_out_hbm.at[b, pl.ds(0, _HALF), :], sem_a)

    cp_b.wait()
    lax.fori_loop(jnp.maximum(hi, _HALF), _SEQ, _zero_row, 0)
    cp_o2 = pltpu.async_copy(rows_v.at[pl.ds(_HALF, _HALF), :],
                             emb_out_hbm.at[b, pl.ds(_HALF, _HALF), :], sem_b)

    cp_mask.wait()
    cp_type.wait()
    cp_word.wait()
    cp_o1.wait()
    cp_o2.wait()


@jax.jit
def kernel(flat_ids, cu_seqlens, flat_emb):
    mesh = plsc.VectorSubcoreMesh(core_axis_name="c", subcore_axis_name="s",
                                  num_cores=1)
    out_type = (
        jax.ShapeDtypeStruct((_B, _SEQ), jnp.int32),
        jax.ShapeDtypeStruct((_B, _SEQ), jnp.int32),
        jax.ShapeDtypeStruct((_B, _SEQ), jnp.int32),
        jax.ShapeDtypeStruct((_B, _SEQ, _D), jnp.float32),
    )
    run = pl.kernel(
        _body,
        out_type=out_type,
        mesh=mesh,
        scratch_types=[
            pltpu.VMEM((32,), jnp.int32),
            pltpu.VMEM((_SEQ,), jnp.int32),
            pltpu.VMEM((_SEQ, _D), jnp.float32),
            pltpu.VMEM((_SEQ,), jnp.int32),
            pltpu.VMEM((_SEQ,), jnp.int32),
            pltpu.VMEM((_SEQ,), jnp.int32),
            pltpu.VMEM((_SEQ,), jnp.int32),
            pltpu.SemaphoreType.DMA,
            pltpu.SemaphoreType.DMA,
            pltpu.SemaphoreType.DMA,
            pltpu.SemaphoreType.DMA,
        ],
    )
    return run(flat_ids.astype(jnp.int32), cu_seqlens.astype(jnp.int32),
               flat_emb)
